# baseline (speedup 1.0000x reference)
import functools

import jax
import jax.numpy as jnp
from jax import lax
from jax.experimental import pallas as pl
from jax.experimental.pallas import tpu as pltpu
from jax.experimental.pallas import tpu_sc as plsc

NC = 2
NS = 16
NW = NC * NS

CH = 32


def kernel(inputs, table):
    B, H = inputs.shape
    V, D = table.shape
    RB = D // 8
    CB = B // 128
    rows_per_w = B // NW
    n_chunks = rows_per_w // CH
    mesh = plsc.VectorSubcoreMesh(core_axis_name="c", subcore_axis_name="s")

    @functools.partial(
        pl.kernel,
        out_type=jax.ShapeDtypeStruct((H, RB, CB, 8, 128), jnp.float32),
        mesh=mesh,
        scratch_types=[
            pltpu.VMEM((CH, H), jnp.int32),
            pltpu.VMEM((CH, H, D), jnp.float32),
            pltpu.VMEM((H, RB, 8, CH), jnp.float32),
            pltpu.SemaphoreType.DMA,
        ],
        compiler_params=pltpu.CompilerParams(
            use_tc_tiling_on_sc=False, needs_layout_passes=False
        ),
    )
    def k(table_hbm, idx_hbm, out_hbm, idx_v, rows_v, buf2, sem):
        wid = lax.axis_index("s") * NC + lax.axis_index("c")
        row_base = wid * rows_per_w
        iota = lax.iota(jnp.int32, 16)

        @pl.loop(0, n_chunks)
        def _chunk(i):
            b0 = pl.multiple_of(row_base + i * CH, CH)
            pltpu.sync_copy(idx_hbm.at[pl.ds(b0, CH)], idx_v)

            @pl.loop(0, CH)
            def _fire(r):
                pltpu.async_copy(table_hbm.at[idx_v.at[r]], rows_v.at[r], sem)

            @pl.loop(0, CH)
            def _drain(r):
                pltpu.make_async_copy(
                    table_hbm.at[idx_v.at[r]], rows_v.at[r], sem
                ).wait()

            for bb in range(CH // 16):
                for d in range(D):

                    @pl.loop(0, H, unroll=10)
                    def _h(h, bb=bb, d=d):
                        ib = iota + bb * 16
                        ih = jnp.full((16,), h, jnp.int32)
                        idv = jnp.full((16,), d, jnp.int32)
                        x = plsc.load_gather(rows_v, [ib, ih, idv])
                        buf2[h, d // 8, d % 8, pl.ds(bb * 16, 16)] = x

            cb = lax.div(b0, 128)
            cc0 = pl.multiple_of(lax.rem(b0, 128), CH)
            pltpu.sync_copy(buf2, out_hbm.at[:, :, cb, :, pl.ds(cc0, CH)])

    out5 = k(table, inputs)
    return out5.transpose(2, 4, 0, 1, 3).reshape(B, H, D)


# speedup vs baseline: 1.6841x; 1.6841x over previous
import functools

import jax
import jax.numpy as jnp
from jax import lax
from jax.experimental import pallas as pl
from jax.experimental.pallas import tpu as pltpu
from jax.experimental.pallas import tpu_sc as plsc

NC = 2
NS = 16
NW = NC * NS

CH = 32


def kernel(inputs, table):
    B, H = inputs.shape
    V, D = table.shape
    RB = D // 8
    CB = B // 128
    rows_per_w = B // NW
    n_chunks = rows_per_w // CH
    mesh = plsc.VectorSubcoreMesh(core_axis_name="c", subcore_axis_name="s")

    @functools.partial(
        pl.kernel,
        out_type=jax.ShapeDtypeStruct((H, RB, CB, 8, 128), jnp.float32),
        mesh=mesh,
        scratch_types=[
            pltpu.VMEM((CH, H), jnp.int32),
            pltpu.VMEM((CH, H, D), jnp.float32),
            pltpu.VMEM((H, RB, 8, CH + 1), jnp.float32),
            pltpu.SemaphoreType.DMA,
        ],
        compiler_params=pltpu.CompilerParams(
            use_tc_tiling_on_sc=False, needs_layout_passes=False
        ),
    )
    def k(table_hbm, idx_hbm, out_hbm, idx_v, rows_v, buf2, sem):
        wid = lax.axis_index("s") * NC + lax.axis_index("c")
        row_base = wid * rows_per_w
        iota = lax.iota(jnp.int32, 16)
        irb0 = iota // 8
        irb2 = irb0 + 2
        irr = iota % 8

        @pl.loop(0, n_chunks)
        def _chunk(i):
            b0 = pl.multiple_of(row_base + i * CH, CH)
            pltpu.sync_copy(idx_hbm.at[pl.ds(b0, CH)], idx_v)

            @pl.loop(0, CH)
            def _fire(r):
                pltpu.async_copy(table_hbm.at[idx_v.at[r]], rows_v.at[r], sem)

            @pl.loop(0, CH)
            def _drain(r):
                pltpu.make_async_copy(table_hbm.at[idx_v.at[r]], rows_v.at[r], sem).wait()

            @pl.loop(0, CH)
            def _b(b):
                ibb = jnp.full((16,), b, jnp.int32)

                @pl.loop(0, H, unroll=10)
                def _h(h, ibb=ibb):
                    ih = jnp.full((16,), h, jnp.int32)
                    x0 = rows_v[b, h, pl.ds(0, 16)]
                    x1 = rows_v[b, h, pl.ds(16, 16)]
                    plsc.store_scatter(buf2, [ih, irb0, irr, ibb], x0)
                    plsc.store_scatter(buf2, [ih, irb2, irr, ibb], x1)

            cb = lax.div(b0, 128)
            cc0 = pl.multiple_of(lax.rem(b0, 128), CH)
            pltpu.sync_copy(buf2.at[:, :, :, pl.ds(0, CH)], out_hbm.at[:, :, cb, :, pl.ds(cc0, CH)])

    out5 = k(table, inputs)
    return out5.transpose(2, 4, 0, 1, 3).reshape(B, H, D)


# speedup vs baseline: 1.7184x; 1.0204x over previous
import functools

import jax
import jax.numpy as jnp
from jax import lax
from jax.experimental import pallas as pl
from jax.experimental.pallas import tpu as pltpu
from jax.experimental.pallas import tpu_sc as plsc

NC = 2
NS = 16
NW = NC * NS

CH = 32


def kernel(inputs, table):
    B, H = inputs.shape
    V, D = table.shape
    RB = D // 8
    CB = B // 128
    rows_per_w = B // NW
    n_chunks = rows_per_w // CH
    mesh = plsc.VectorSubcoreMesh(core_axis_name="c", subcore_axis_name="s")

    @functools.partial(
        pl.kernel,
        out_type=jax.ShapeDtypeStruct((H, RB, CB, 8, 128), jnp.float32),
        mesh=mesh,
        scratch_types=[
            pltpu.VMEM((CH, H), jnp.int32),
            pltpu.VMEM((CH, H, D), jnp.float32),
            pltpu.VMEM((H, RB, 8, CH + 1), jnp.float32),
            pltpu.SemaphoreType.DMA,
        ],
        compiler_params=pltpu.CompilerParams(
            use_tc_tiling_on_sc=False, needs_layout_passes=False
        ),
    )
    def k(table_hbm, idx_hbm, out_hbm, idx_v, rows_v, buf2, sem):
        wid = lax.axis_index("s") * NC + lax.axis_index("c")
        row_base = wid * rows_per_w
        iota = lax.iota(jnp.int32, 16)
        irb0 = iota // 8
        irb2 = irb0 + 2
        irr = iota % 8

        b_pro = pl.multiple_of(row_base, CH)
        pltpu.sync_copy(idx_hbm.at[pl.ds(b_pro, CH)], idx_v)

        @pl.loop(0, CH)
        def _fire0(r):
            pltpu.async_copy(table_hbm.at[idx_v.at[r]], rows_v.at[r], sem)

        @pl.loop(0, n_chunks)
        def _chunk(i):
            b0 = pl.multiple_of(row_base + i * CH, CH)

            @pl.loop(0, CH)
            def _drain(r):
                pltpu.make_async_copy(table_hbm.at[idx_v.at[r]], rows_v.at[r], sem).wait()

            @pl.loop(0, CH)
            def _b(b):
                ibb = jnp.full((16,), b, jnp.int32)

                @pl.loop(0, H, unroll=10)
                def _h(h, ibb=ibb):
                    ih = jnp.full((16,), h, jnp.int32)
                    x0 = rows_v[b, h, pl.ds(0, 16)]
                    x1 = rows_v[b, h, pl.ds(16, 16)]
                    plsc.store_scatter(buf2, [ih, irb0, irr, ibb], x0)
                    plsc.store_scatter(buf2, [ih, irb2, irr, ibb], x1)

            @pl.when(i < n_chunks - 1)
            def _():
                b1 = pl.multiple_of(row_base + (i + 1) * CH, CH)
                pltpu.sync_copy(idx_hbm.at[pl.ds(b1, CH)], idx_v)

                @pl.loop(0, CH)
                def _fire(r):
                    pltpu.async_copy(table_hbm.at[idx_v.at[r]], rows_v.at[r], sem)

            cb = lax.div(b0, 128)
            cc0 = pl.multiple_of(lax.rem(b0, 128), CH)
            pltpu.sync_copy(buf2.at[:, :, :, pl.ds(0, CH)], out_hbm.at[:, :, cb, :, pl.ds(cc0, CH)])

    out5 = k(table, inputs)
    return out5.transpose(2, 4, 0, 1, 3).reshape(B, H, D)
